# Initial kernel scaffold; baseline (speedup 1.0000x reference)
import jax
import jax.numpy as jnp
from jax.experimental import pallas as pl


def kernel(x, M_init, eta_w, eta_b, alpha_w, alpha_b, gate_w, gate_b, vg_w1, vg_w2):
    raise NotImplementedError("write your pallas kernel here")


# speedup vs baseline: 4.2421x; 4.2421x over previous
import jax
import jax.numpy as jnp
from jax.experimental import pallas as pl
from jax.experimental.pallas import tpu as pltpu

B, S, D = 4, 4096, 1024
CHUNK = 64
CPB = 8
MAX_LR = 0.2
MIN_DECAY = 0.5
MAX_NORM = 30.0
NORM_EPS = 1e-5

_DNT = (((1,), (1,)), ((), ()))
_DN0 = (((0,), (0,)), ((), ()))


def _hope_kernel(x_ref, m0_ref, ew_ref, eb_ref, aw_ref, ab_ref,
                 gw_ref, gb_ref, w1_ref, w2_ref, out_ref, mfin_ref, s_ref):
    c = pl.program_id(1)

    @pl.when(c == 0)
    def _init():
        m0 = m0_ref[...]
        mfin_ref[0] = m0
        s_ref[0] = jnp.sum(m0 * m0)

    ew = ew_ref[...]
    aw = aw_ref[...]
    gw = gw_ref[...]
    eb = eb_ref[0]
    ab = ab_ref[0]
    gb = gb_ref[0]
    w1 = w1_ref[...]
    w2 = w2_ref[...]
    ones_rep = jnp.full((8, 128), 0.125, jnp.float32)

    for k in range(CPB):
        chunk = x_ref[0, k * CHUNK:(k + 1) * CHUNK, :]
        M = mfin_ref[0]
        S_sc = s_ref[0]

        eta = jnp.mean(jax.nn.sigmoid(
            jnp.sum(chunk * ew, axis=1, keepdims=True) + eb)) * MAX_LR
        alpha = MIN_DECAY + jnp.mean(jax.nn.sigmoid(
            jnp.sum(chunk * aw, axis=1, keepdims=True) + ab)) * (1.0 - MIN_DECAY)

        outputs = jax.lax.dot_general(chunk, M, _DNT,
                                      preferred_element_type=jnp.float32)

        nrm = jnp.sqrt(jnp.sum(chunk * chunk, axis=1, keepdims=True))
        keys = chunk / jnp.maximum(nrm, NORM_EPS)
        k_mean = jnp.mean(keys, axis=0, keepdims=True)

        h1 = jax.lax.dot_general(outputs, w1, _DNT,
                                 preferred_element_type=jnp.float32)
        h1 = h1 * jax.nn.sigmoid(h1)
        h = jax.lax.dot_general(h1, w2, _DNT,
                                preferred_element_type=jnp.float32) + outputs
        out_ref[0, k * CHUNK:(k + 1) * CHUNK, :] = h

        v_t = jnp.mean(h, axis=0, keepdims=True)
        Mk = jax.lax.dot_general(k_mean, M, _DNT,
                                 preferred_element_type=jnp.float32)
        err = v_t - Mk

        gate = jax.nn.sigmoid(jnp.sum(k_mean * gw) + gb)
        gA = gate * alpha + (1.0 - gate)
        gB = gate * eta

        t_cross = jnp.sum(err * Mk)
        r_sq = jnp.sum(err * err) * jnp.sum(k_mean * k_mean)
        fro2 = gA * gA * S_sc + 2.0 * gA * gB * t_cross + gB * gB * r_sq
        scale = jnp.minimum(MAX_NORM / (jnp.sqrt(fro2) + 1e-6), 1.0)

        cA = scale * gA
        cB = scale * gB

        err8 = jnp.broadcast_to(err, (8, D))
        err_rep = jax.lax.dot_general(err8, ones_rep, _DN0,
                                      preferred_element_type=jnp.float32)
        err_full = jnp.concatenate([err_rep] * 8, axis=1)
        mfin_ref[0] = cA * M + (cB * err_full) * k_mean
        s_ref[0] = scale * scale * fro2


def kernel(x, M_init, eta_w, eta_b, alpha_w, alpha_b, gate_w, gate_b,
           vg_w1, vg_w2):
    bs = CHUNK * CPB
    grid = (B, S // bs)

    in_specs = [
        pl.BlockSpec((1, bs, D), lambda b, c: (b, c, 0)),
        pl.BlockSpec((D, D), lambda b, c: (0, 0)),
        pl.BlockSpec((1, D), lambda b, c: (0, 0)),
        pl.BlockSpec(memory_space=pltpu.SMEM),
        pl.BlockSpec((1, D), lambda b, c: (0, 0)),
        pl.BlockSpec(memory_space=pltpu.SMEM),
        pl.BlockSpec((1, D), lambda b, c: (0, 0)),
        pl.BlockSpec(memory_space=pltpu.SMEM),
        pl.BlockSpec((D, D), lambda b, c: (0, 0)),
        pl.BlockSpec((D, D), lambda b, c: (0, 0)),
    ]
    out_specs = [
        pl.BlockSpec((1, bs, D), lambda b, c: (b, c, 0)),
        pl.BlockSpec((1, D, D), lambda b, c: (b, 0, 0)),
    ]
    out_shape = [
        jax.ShapeDtypeStruct((B, S, D), jnp.float32),
        jax.ShapeDtypeStruct((B, D, D), jnp.float32),
    ]

    final_out, m_final = pl.pallas_call(
        _hope_kernel,
        grid=grid,
        in_specs=in_specs,
        out_specs=out_specs,
        out_shape=out_shape,
        scratch_shapes=[pltpu.SMEM((1,), jnp.float32)],
        compiler_params=pltpu.CompilerParams(
            dimension_semantics=("parallel", "arbitrary"),
            vmem_limit_bytes=64 * 1024 * 1024,
        ),
    )(x, M_init, eta_w.reshape(1, D), eta_b, alpha_w.reshape(1, D), alpha_b,
      gate_w.reshape(1, D), gate_b, vg_w1, vg_w2)
    return final_out, m_final


# speedup vs baseline: 4.2748x; 1.0077x over previous
import jax
import jax.numpy as jnp
from jax.experimental import pallas as pl
from jax.experimental.pallas import tpu as pltpu

B, S, D = 4, 4096, 1024
CHUNK = 64
CPB = 8
MAX_LR = 0.2
MIN_DECAY = 0.5
MAX_NORM = 30.0
NORM_EPS = 1e-5

_DNT = (((1,), (1,)), ((), ()))
_DN0 = (((0,), (0,)), ((), ()))


def _hope_kernel(x_ref, m0_ref, ew_ref, eb_ref, aw_ref, ab_ref,
                 gw_ref, gb_ref, w1_ref, w2_ref, out_ref, mfin_ref,
                 s_ref, mbf_ref):
    c = pl.program_id(1)

    @pl.when(c == 0)
    def _init():
        m0 = m0_ref[...]
        mfin_ref[0] = m0
        mbf_ref[...] = m0.astype(jnp.bfloat16)
        s_ref[0] = jnp.sum(m0 * m0)

    ew = ew_ref[...]
    aw = aw_ref[...]
    gw = gw_ref[...]
    eb = eb_ref[0]
    ab = ab_ref[0]
    gb = gb_ref[0]
    w1 = w1_ref[...]
    w2 = w2_ref[...]
    ones_rep = jnp.full((8, 128), 0.125, jnp.float32)

    for k in range(CPB):
        chunk = x_ref[0, k * CHUNK:(k + 1) * CHUNK, :]
        chunk_b = chunk.astype(jnp.bfloat16)
        M = mfin_ref[0]
        Mb = mbf_ref[...]
        S_sc = s_ref[0]

        eta = jnp.mean(jax.nn.sigmoid(
            jnp.sum(chunk * ew, axis=1, keepdims=True) + eb)) * MAX_LR
        alpha = MIN_DECAY + jnp.mean(jax.nn.sigmoid(
            jnp.sum(chunk * aw, axis=1, keepdims=True) + ab)) * (1.0 - MIN_DECAY)

        outputs = jax.lax.dot_general(chunk_b, Mb, _DNT,
                                      preferred_element_type=jnp.float32)

        nrm = jnp.sqrt(jnp.sum(chunk * chunk, axis=1, keepdims=True))
        keys = chunk / jnp.maximum(nrm, NORM_EPS)
        k_mean = jnp.mean(keys, axis=0, keepdims=True)

        h1 = jax.lax.dot_general(outputs.astype(jnp.bfloat16), w1, _DNT,
                                 preferred_element_type=jnp.float32)
        h1 = h1 * jax.nn.sigmoid(h1)
        h = jax.lax.dot_general(h1.astype(jnp.bfloat16), w2, _DNT,
                                preferred_element_type=jnp.float32) + outputs
        out_ref[0, k * CHUNK:(k + 1) * CHUNK, :] = h

        v_t = jnp.mean(h, axis=0, keepdims=True)
        Mk = jax.lax.dot_general(k_mean.astype(jnp.bfloat16), Mb, _DNT,
                                 preferred_element_type=jnp.float32)
        err = v_t - Mk

        gate = jax.nn.sigmoid(jnp.sum(k_mean * gw) + gb)
        gA = gate * alpha + (1.0 - gate)
        gB = gate * eta

        t_cross = jnp.sum(err * Mk)
        r_sq = jnp.sum(err * err) * jnp.sum(k_mean * k_mean)
        fro2 = gA * gA * S_sc + 2.0 * gA * gB * t_cross + gB * gB * r_sq
        scale = jnp.minimum(MAX_NORM / (jnp.sqrt(fro2) + 1e-6), 1.0)

        cA = scale * gA
        cB = scale * gB

        err8 = jnp.broadcast_to(cB * err, (8, D))
        err_rep = jax.lax.dot_general(err8, ones_rep, _DN0,
                                      preferred_element_type=jnp.float32)
        err_full = jnp.concatenate([err_rep] * 8, axis=1)
        m_new = cA * M + err_full * k_mean
        mfin_ref[0] = m_new
        mbf_ref[...] = m_new.astype(jnp.bfloat16)
        s_ref[0] = scale * scale * fro2


def kernel(x, M_init, eta_w, eta_b, alpha_w, alpha_b, gate_w, gate_b,
           vg_w1, vg_w2):
    bs = CHUNK * CPB
    grid = (B, S // bs)

    in_specs = [
        pl.BlockSpec((1, bs, D), lambda b, c: (b, c, 0)),
        pl.BlockSpec((D, D), lambda b, c: (0, 0)),
        pl.BlockSpec((1, D), lambda b, c: (0, 0)),
        pl.BlockSpec(memory_space=pltpu.SMEM),
        pl.BlockSpec((1, D), lambda b, c: (0, 0)),
        pl.BlockSpec(memory_space=pltpu.SMEM),
        pl.BlockSpec((1, D), lambda b, c: (0, 0)),
        pl.BlockSpec(memory_space=pltpu.SMEM),
        pl.BlockSpec((D, D), lambda b, c: (0, 0)),
        pl.BlockSpec((D, D), lambda b, c: (0, 0)),
    ]
    out_specs = [
        pl.BlockSpec((1, bs, D), lambda b, c: (b, c, 0)),
        pl.BlockSpec((1, D, D), lambda b, c: (b, 0, 0)),
    ]
    out_shape = [
        jax.ShapeDtypeStruct((B, S, D), jnp.float32),
        jax.ShapeDtypeStruct((B, D, D), jnp.float32),
    ]

    final_out, m_final = pl.pallas_call(
        _hope_kernel,
        grid=grid,
        in_specs=in_specs,
        out_specs=out_specs,
        out_shape=out_shape,
        scratch_shapes=[pltpu.SMEM((1,), jnp.float32),
                        pltpu.VMEM((D, D), jnp.bfloat16)],
        compiler_params=pltpu.CompilerParams(
            dimension_semantics=("parallel", "arbitrary"),
            vmem_limit_bytes=64 * 1024 * 1024,
        ),
    )(x, M_init, eta_w.reshape(1, D), eta_b, alpha_w.reshape(1, D), alpha_b,
      gate_w.reshape(1, D), gate_b, vg_w1.astype(jnp.bfloat16),
      vg_w2.astype(jnp.bfloat16))
    return final_out, m_final
